# Initial kernel scaffold; baseline (speedup 1.0000x reference)
import jax
import jax.numpy as jnp
from jax.experimental import pallas as pl


def kernel(weights):
    raise NotImplementedError("write your pallas kernel here")


# speedup vs baseline: 2.0529x; 2.0529x over previous
import functools

import jax
import jax.numpy as jnp
from jax import lax
from jax.experimental import pallas as pl
from jax.experimental.pallas import tpu as pltpu
from jax.experimental.pallas import tpu_sc as plsc

_BATCH = 1024
_V = 200
_U1 = 101
_NORM = 18.8736
_L = 16
_NCHUNK = _BATCH // _L
_NWORK = 32
_CPW = _NCHUNK // _NWORK


def _sc_body(w_hbm, g_hbm, t_hbm, seq_hbm, size_hbm,
             wv0, wv1, gv0, gv1, tv, matched, seqbuf, sizebuf,
             sw0, sw1, sg0, sg1):
    wid = lax.axis_index("s") * 2 + lax.axis_index("c")
    lane = lax.broadcasted_iota(jnp.int32, (_L,), 0)
    zeros = jnp.zeros((_L,), jnp.float32)
    ones = jnp.ones((_L,), jnp.float32)

    def chunk_body(k, _):
        c = wid * _CPW + k
        for u in range(_U1):
            matched[u] = zeros
        pltpu.sync_copy(t_hbm.at[c], tv)
        pltpu.make_async_copy(w_hbm.at[c, 0], wv0, sw0).start()
        pltpu.make_async_copy(g_hbm.at[c, 0], gv0, sg0).start()
        pltpu.make_async_copy(w_hbm.at[c, 1], wv1, sw1).start()
        pltpu.make_async_copy(g_hbm.at[c, 1], gv1, sg1).start()

        def v_body(i, size):
            for b in range(2):
                v = 2 * i + b
                sw = (sw0, sw1)[b]
                sg = (sg0, sg1)[b]
                wv = (wv0, wv1)[b]
                gv = (gv0, gv1)[b]
                pltpu.make_async_copy(w_hbm.at[c, v], wv, sw).wait()
                pltpu.make_async_copy(g_hbm.at[c, v], gv, sg).wait()
                t = tv[...]
                best = jnp.full((_L,), -1e30, jnp.float32)
                bidx = jnp.zeros((_L,), jnp.int32)
                wbest = zeros
                anyv = jnp.zeros((_L,), jnp.bool_)
                for u in range(1, _U1):
                    wu = wv[u]
                    gu = gv[u]
                    mu = matched[u]
                    a = wu * _NORM
                    setm = (a > 0.0) & ((a + 1.0) >= t) & (mu == 0.0)
                    upd = setm & (gu > best)
                    best = jnp.where(upd, gu, best)
                    bidx = jnp.where(upd, jnp.int32(u), bidx)
                    wbest = jnp.where(upd, wu, wbest)
                    anyv = anyv | setm
                sel = jnp.where(anyv, bidx, jnp.int32(0))
                size = size + jnp.where(anyv, wbest, 0.0)
                plsc.store_scatter(matched, [sel, lane], ones, mask=anyv)
                plsc.store_scatter(seqbuf, [jnp.full((_L,), v, jnp.int32), lane], sel)
                nv = v + 2

                @pl.when(nv < _V)
                def _():
                    pltpu.make_async_copy(w_hbm.at[c, nv], wv, sw).start()
                    pltpu.make_async_copy(g_hbm.at[c, nv], gv, sg).start()
            return size

        size = lax.fori_loop(0, _V // 2, v_body, zeros)
        sizebuf[...] = -size
        pltpu.sync_copy(seqbuf, seq_hbm.at[c])
        pltpu.sync_copy(sizebuf, size_hbm.at[c])
        return 0

    lax.fori_loop(0, _CPW, chunk_body, 0)


@jax.jit
def _sc_call(w_t, g_t, t_t):
    mesh = plsc.VectorSubcoreMesh(core_axis_name="c", subcore_axis_name="s")
    f = pl.kernel(
        _sc_body,
        out_type=(
            jax.ShapeDtypeStruct((_NCHUNK, _V, _L), jnp.int32),
            jax.ShapeDtypeStruct((_NCHUNK, _L), jnp.float32),
        ),
        mesh=mesh,
        scratch_types=[
            pltpu.VMEM((_U1, _L), jnp.float32),
            pltpu.VMEM((_U1, _L), jnp.float32),
            pltpu.VMEM((_U1, _L), jnp.float32),
            pltpu.VMEM((_U1, _L), jnp.float32),
            pltpu.VMEM((_L,), jnp.float32),
            pltpu.VMEM((_U1, _L), jnp.float32),
            pltpu.VMEM((_V, _L), jnp.int32),
            pltpu.VMEM((_L,), jnp.float32),
            pltpu.SemaphoreType.DMA,
            pltpu.SemaphoreType.DMA,
            pltpu.SemaphoreType.DMA,
            pltpu.SemaphoreType.DMA,
        ],
        compiler_params=pltpu.CompilerParams(needs_layout_passes=False),
    )
    return f(w_t, g_t, t_t)


def kernel(weights):
    kt, kg = jax.random.split(jax.random.key(42))
    t = jnp.exp(jax.random.randint(kt, (_BATCH, 1), 1, 3).astype(jnp.float32))
    gumbel = jax.random.gumbel(kg, (_V, _BATCH, _U1), dtype=jnp.float32)
    w_t = weights.reshape(_NCHUNK, _L, _V, _U1).transpose(0, 2, 3, 1)
    g_t = gumbel.reshape(_V, _NCHUNK, _L, _U1).transpose(1, 0, 3, 2)
    t_t = t.reshape(_NCHUNK, _L)
    seq, neg_size = _sc_call(w_t, g_t, t_t)
    return neg_size.reshape(_BATCH), seq.transpose(0, 2, 1).reshape(_BATCH, _V)
